# baseline (speedup 1.0000x reference)
import functools

import jax
import jax.numpy as jnp
from jax import lax
from jax.experimental import pallas as pl
from jax.experimental.pallas import tpu as pltpu
from jax.experimental.pallas import tpu_sc as plsc

HIDDEN = 128
NUM_WORKERS = 32
NBUF = 5


def kernel(input_ids, weight):
    B, S = input_ids.shape
    bpw = B // NUM_WORKERS
    idx_t = input_ids.astype(jnp.int32).T

    mesh = plsc.VectorSubcoreMesh(core_axis_name="c", subcore_axis_name="s")

    @functools.partial(
        pl.kernel,
        mesh=mesh,
        out_type=jax.ShapeDtypeStruct((S, B, HIDDEN), jnp.float32),
        compiler_params=pltpu.CompilerParams(use_tc_tiling_on_sc=True),
        scratch_types=[
            pltpu.VMEM((S, bpw), jnp.int32),
            pltpu.VMEM((NBUF, bpw, HIDDEN), jnp.float32),
            [pltpu.SemaphoreType.DMA] * NBUF,
            [pltpu.SemaphoreType.DMA] * NBUF,
            pltpu.SemaphoreType.DMA,
        ],
    )
    def emb(table_hbm, idx_hbm, out_hbm, idx_v, rows_v, sem_in, sem_out,
            sem_idx):
        wid = lax.axis_index("s") * 2 + lax.axis_index("c")
        base = wid * bpw

        head = 8
        pltpu.sync_copy(idx_hbm.at[pl.ds(0, head), pl.ds(base, bpw)],
                        idx_v.at[pl.ds(0, head)])
        rest = pltpu.make_async_copy(
            idx_hbm.at[pl.ds(head, S - head), pl.ds(base, bpw)],
            idx_v.at[pl.ds(head, S - head)], sem_idx)
        rest.start()

        def start_gather(s, b):
            pltpu.make_async_copy(
                table_hbm.at[idx_v.at[s]], rows_v.at[b], sem_in[b]).start()

        def wait_gather(b):
            pltpu.make_async_copy(
                table_hbm.at[idx_v.at[0]], rows_v.at[b], sem_in[b]).wait()

        def start_write(s, b):
            pltpu.make_async_copy(
                rows_v.at[b], out_hbm.at[s, pl.ds(base, bpw)],
                sem_out[b]).start()

        def wait_write(b):
            pltpu.make_async_copy(
                rows_v.at[b], out_hbm.at[0, pl.ds(base, bpw)],
                sem_out[b]).wait()

        start_gather(0, 0)
        start_gather(1, 1)
        for s0 in (0, 1, 2):
            start_gather(s0 + 2, s0 + 2)
            wait_gather(s0)
            start_write(s0, s0)
        rest.wait()

        def step(s, b):
            nb = (b + 2) % NBUF
            wait_write(nb)
            start_gather(s + 2, nb)
            wait_gather(b)
            start_write(s, b)

        def body(k, carry):
            for off in range(NBUF):
                s = NBUF * k + 3 + off
                step(s, (3 + off) % NBUF)
            return carry

        lax.fori_loop(0, (S - 5) // NBUF, body, 0)

        wait_gather((S - 2) % NBUF)
        start_write(S - 2, (S - 2) % NBUF)
        wait_gather((S - 1) % NBUF)
        start_write(S - 1, (S - 1) % NBUF)
        for b in range(NBUF):
            wait_write(b)

    out = emb(weight, idx_t)
    return out.transpose(1, 0, 2)


# speedup vs baseline: 1.0041x; 1.0041x over previous
import functools

import jax
import jax.numpy as jnp
from jax import lax
from jax.experimental import pallas as pl
from jax.experimental.pallas import tpu as pltpu
from jax.experimental.pallas import tpu_sc as plsc

HIDDEN = 128
NUM_WORKERS = 32
NBUF = 5


def kernel(input_ids, weight):
    B, S = input_ids.shape
    bpw = B // NUM_WORKERS
    idx_t = input_ids.astype(jnp.int32).T

    mesh = plsc.VectorSubcoreMesh(core_axis_name="c", subcore_axis_name="s")

    @functools.partial(
        pl.kernel,
        mesh=mesh,
        out_type=jax.ShapeDtypeStruct((S, B, HIDDEN), jnp.float32),
        compiler_params=pltpu.CompilerParams(use_tc_tiling_on_sc=True),
        scratch_types=[
            pltpu.VMEM((S, bpw), jnp.int32),
            pltpu.VMEM((NBUF, bpw, HIDDEN), jnp.float32),
            [pltpu.SemaphoreType.DMA] * NBUF,
            [pltpu.SemaphoreType.DMA] * NBUF,
            pltpu.SemaphoreType.DMA,
        ],
    )
    def emb(table_hbm, idx_hbm, out_hbm, idx_v, rows_v, sem_in, sem_out,
            sem_idx):
        wid = lax.axis_index("s") * 2 + lax.axis_index("c")
        base = wid * bpw

        head = 8
        pltpu.sync_copy(idx_hbm.at[pl.ds(0, head), pl.ds(base, bpw)],
                        idx_v.at[pl.ds(0, head)])
        rest = pltpu.make_async_copy(
            idx_hbm.at[pl.ds(head, S - head), pl.ds(base, bpw)],
            idx_v.at[pl.ds(head, S - head)], sem_idx)
        rest.start()

        def start_gather(s, b):
            pltpu.make_async_copy(
                table_hbm.at[idx_v.at[s]], rows_v.at[b], sem_in[b]).start()

        def wait_gather(b):
            pltpu.make_async_copy(
                table_hbm.at[idx_v.at[0]], rows_v.at[b], sem_in[b]).wait()

        def start_write(s, b):
            pltpu.make_async_copy(
                rows_v.at[b], out_hbm.at[s, pl.ds(base, bpw)],
                sem_out[b]).start()

        def wait_write(b):
            pltpu.make_async_copy(
                rows_v.at[b], out_hbm.at[0, pl.ds(base, bpw)],
                sem_out[b]).wait()

        start_gather(0, 0)
        start_gather(1, 1)
        start_gather(2, 2)
        for s0 in (0, 1):
            start_gather(s0 + 3, s0 + 3)
            wait_gather(s0)
            start_write(s0, s0)
        rest.wait()

        def step(s, b):
            nb = (b + 3) % NBUF
            wait_write(nb)
            start_gather(s + 3, nb)
            wait_gather(b)
            start_write(s, b)

        def body(k, carry):
            for off in range(NBUF):
                s = NBUF * k + 2 + off
                step(s, (2 + off) % NBUF)
            return carry

        lax.fori_loop(0, (S - 5) // NBUF, body, 0)

        for st in (S - 3, S - 2, S - 1):
            wait_gather(st % NBUF)
            start_write(st, st % NBUF)
        for b in range(NBUF):
            wait_write(b)

    out = emb(weight, idx_t)
    return out.transpose(1, 0, 2)


# speedup vs baseline: 1.0123x; 1.0082x over previous
import functools

import jax
import jax.numpy as jnp
from jax import lax
from jax.experimental import pallas as pl
from jax.experimental.pallas import tpu as pltpu
from jax.experimental.pallas import tpu_sc as plsc

HIDDEN = 128
NUM_WORKERS = 32
NBUF = 6


def kernel(input_ids, weight):
    B, S = input_ids.shape
    bpw = B // NUM_WORKERS
    idx_t = input_ids.astype(jnp.int32).T

    mesh = plsc.VectorSubcoreMesh(core_axis_name="c", subcore_axis_name="s")

    @functools.partial(
        pl.kernel,
        mesh=mesh,
        out_type=jax.ShapeDtypeStruct((S, B, HIDDEN), jnp.float32),
        compiler_params=pltpu.CompilerParams(use_tc_tiling_on_sc=True),
        scratch_types=[
            pltpu.VMEM((S, bpw), jnp.int32),
            pltpu.VMEM((NBUF, bpw, HIDDEN), jnp.float32),
            [pltpu.SemaphoreType.DMA] * NBUF,
            [pltpu.SemaphoreType.DMA] * NBUF,
            pltpu.SemaphoreType.DMA,
        ],
    )
    def emb(table_hbm, idx_hbm, out_hbm, idx_v, rows_v, sem_in, sem_out,
            sem_idx):
        wid = lax.axis_index("s") * 2 + lax.axis_index("c")
        base = wid * bpw

        head = 8
        pltpu.sync_copy(idx_hbm.at[pl.ds(0, head), pl.ds(base, bpw)],
                        idx_v.at[pl.ds(0, head)])
        rest = pltpu.make_async_copy(
            idx_hbm.at[pl.ds(head, S - head), pl.ds(base, bpw)],
            idx_v.at[pl.ds(head, S - head)], sem_idx)
        rest.start()

        def start_gather(s, b):
            pltpu.make_async_copy(
                table_hbm.at[idx_v.at[s]], rows_v.at[b], sem_in[b]).start()

        def wait_gather(b):
            pltpu.make_async_copy(
                table_hbm.at[idx_v.at[0]], rows_v.at[b], sem_in[b]).wait()

        def start_write(s, b):
            pltpu.make_async_copy(
                rows_v.at[b], out_hbm.at[s, pl.ds(base, bpw)],
                sem_out[b]).start()

        def wait_write(b):
            pltpu.make_async_copy(
                rows_v.at[b], out_hbm.at[0, pl.ds(base, bpw)],
                sem_out[b]).wait()

        for s0 in (0, 1, 2, 3):
            start_gather(s0, s0)
        for s0 in (0, 1):
            start_gather(s0 + 4, s0 + 4)
            wait_gather(s0)
            start_write(s0, s0)
        rest.wait()

        def step(s, b):
            nb = (b + 4) % NBUF
            wait_write(nb)
            start_gather(s + 4, nb)
            wait_gather(b)
            start_write(s, b)

        def body(k, carry):
            for off in range(NBUF):
                s = NBUF * k + 2 + off
                step(s, (2 + off) % NBUF)
            return carry

        lax.fori_loop(0, (S - 8) // NBUF, body, 0)
        step(S - 6, (S - 6) % NBUF)
        step(S - 5, (S - 5) % NBUF)

        for st in (S - 4, S - 3, S - 2, S - 1):
            wait_gather(st % NBUF)
            start_write(st, st % NBUF)
        for b in range(NBUF):
            wait_write(b)

    out = emb(weight, idx_t)
    return out.transpose(1, 0, 2)
